# baseline (speedup 1.0000x reference)
import jax
import jax.numpy as jnp
from jax.experimental import pallas as pl
from jax.experimental.pallas import tpu as pltpu

_NUM_CLASSES = 100000
_BLOCK_ROWS = 32
_NSLOTS = 4


def _onehot_body(idx_ref, out_ref, *scratch):
    bufs = scratch[:_NSLOTS]
    sems = scratch[_NSLOTS:]
    i = pl.program_id(0)
    n = pl.num_programs(0)

    def copy_for(step, slot):
        return pltpu.make_async_copy(
            bufs[slot],
            out_ref.at[pl.ds(step * _BLOCK_ROWS, _BLOCK_ROWS), :],
            sems[slot],
        )

    for k in range(_NSLOTS):
        @pl.when(jax.lax.rem(i, _NSLOTS) == k)
        def _(k=k):
            @pl.when(i >= _NSLOTS)
            def _():
                copy_for(i - _NSLOTS, k).wait()

            rows = idx_ref[pl.ds(i * _BLOCK_ROWS, _BLOCK_ROWS), :]
            cols = jax.lax.broadcasted_iota(
                jnp.int32, (_BLOCK_ROWS, _NUM_CLASSES), 1
            )
            bufs[k][:, :] = (rows == cols).astype(jnp.float32)
            copy_for(i, k).start()

    @pl.when(i == n - 1)
    def _():
        for d in range(_NSLOTS - 1, -1, -1):
            step = i - d
            for k in range(_NSLOTS):
                @pl.when(jax.lax.rem(step, _NSLOTS) == k)
                def _(step=step, k=k):
                    copy_for(step, k).wait()


def kernel(idx):
    b = idx.shape[0]
    idx2 = idx.astype(jnp.int32).reshape(b, 1)
    grid = (b // _BLOCK_ROWS,)
    return pl.pallas_call(
        _onehot_body,
        grid=grid,
        in_specs=[pl.BlockSpec((b, 1), lambda i: (0, 0))],
        out_specs=pl.BlockSpec(memory_space=pl.ANY),
        out_shape=jax.ShapeDtypeStruct((b, _NUM_CLASSES), jnp.float32),
        scratch_shapes=(
            [pltpu.VMEM((_BLOCK_ROWS, _NUM_CLASSES), jnp.float32)] * _NSLOTS
            + [pltpu.SemaphoreType.DMA] * _NSLOTS
        ),
    )(idx2)
